# baseline (speedup 1.0000x reference)
import jax
import jax.numpy as jnp
from jax import lax
from jax.experimental import pallas as pl
from jax.experimental.pallas import tpu as pltpu
from jax.experimental.pallas import tpu_sc as plsc

D = 64
B_TOTAL = 4096 * 200

NUM_CORES = 2
NUM_SUBCORES = 16
NW = NUM_CORES * NUM_SUBCORES
B_PER_W = B_TOTAL // NW

GROUP = 128
G_PER_W = B_PER_W // GROUP
K = 4
CHUNK = K * GROUP
N_CHUNKS = B_PER_W // CHUNK
NBUF = 2


def _body(x_hbm, table_hbm, out_hbm, idx_all, pos2d, rows0, rows1,
          gsem0, gsem1, wsem0, wsem1):
    wid = lax.axis_index("s") * NUM_CORES + lax.axis_index("c")
    base = pl.multiple_of(wid * B_PER_W, 8)

    pltpu.sync_copy(x_hbm.at[pl.ds(base, B_PER_W)], idx_all)

    def fill(i, carry):
        r = i // 8
        l = i % 8
        pos2d[r, pl.ds(l * 16, 16)] = (
            base + r * GROUP + l * 16 + lax.iota(jnp.int32, 16)
        )
        return carry

    lax.fori_loop(0, G_PER_W * 8, fill, 0)

    rows = (rows0, rows1)
    gsem = (gsem0, gsem1)
    wsem = (wsem0, wsem1)

    def fire_gather(c, p):
        pltpu.async_copy(
            table_hbm.at[idx_all.at[pl.ds(c * CHUNK, CHUNK)]],
            rows[p],
            gsem[p],
        )

    def wait_gather(c, p):
        pltpu.make_async_copy(
            table_hbm.at[idx_all.at[pl.ds(c * CHUNK, CHUNK)]],
            rows[p],
            gsem[p],
        ).wait()

    def fire_writeback(c, p):
        for j in range(K):
            pltpu.async_copy(
                rows[p].at[pl.ds(j * GROUP, GROUP)],
                out_hbm.at[pos2d.at[c * K + j]],
                wsem[p],
            )

    def wait_writeback(c, p):
        for j in range(K):
            pltpu.make_async_copy(
                rows[p].at[pl.ds(j * GROUP, GROUP)],
                out_hbm.at[pos2d.at[c * K + j]],
                wsem[p],
            ).wait()

    fire_gather(0, 0)
    fire_gather(1, 1)

    def step(c2, carry):
        c = c2 * NBUF
        wait_gather(c, 0)
        fire_writeback(c, 0)
        wait_writeback(c, 0)
        fire_gather(c + 2, 0)
        wait_gather(c + 1, 1)
        fire_writeback(c + 1, 1)
        wait_writeback(c + 1, 1)
        fire_gather(c + 3, 1)
        return carry

    lax.fori_loop(0, (N_CHUNKS - NBUF) // NBUF, step, 0)

    c = N_CHUNKS - 2
    wait_gather(c, 0)
    fire_writeback(c, 0)
    wait_gather(c + 1, 1)
    fire_writeback(c + 1, 1)
    wait_writeback(c, 0)
    wait_writeback(c + 1, 1)


@jax.jit
def kernel(x, table):
    xf = x.reshape(-1)
    mesh = plsc.VectorSubcoreMesh(
        core_axis_name="c", subcore_axis_name="s"
    )
    out = pl.kernel(
        _body,
        out_type=jax.ShapeDtypeStruct((B_TOTAL, D), jnp.float32),
        mesh=mesh,
        compiler_params=pltpu.CompilerParams(use_tc_tiling_on_sc=False),
        scratch_types=[
            pltpu.VMEM((B_PER_W,), jnp.int32),
            pltpu.VMEM((G_PER_W, GROUP), jnp.int32),
            pltpu.VMEM((CHUNK, D), jnp.float32),
            pltpu.VMEM((CHUNK, D), jnp.float32),
            pltpu.SemaphoreType.DMA,
            pltpu.SemaphoreType.DMA,
            pltpu.SemaphoreType.DMA,
            pltpu.SemaphoreType.DMA,
        ],
    )(xf, table)
    return out.reshape(x.shape[0], x.shape[1], D)
